# baseline (speedup 1.0000x reference)
import functools

import jax
import jax.numpy as jnp
from jax.experimental import pallas as pl
from jax.experimental.pallas import tpu as pltpu
from jax.experimental.pallas import tpu_sc as plsc

_EPSILON = 0.05
_SRC_ALPHA = 2.0
_TGT_ALPHA = 2.0
_EVENT_ALPHA = 2.0
_ROW_BLOCK = 64

_KEY_BASE = -1107296257
_KEY_BITS = 28


def _float_key(f):
    b = jax.lax.bitcast_convert_type(f, jnp.int32)
    return b ^ (jnp.right_shift(b, 31) & jnp.int32(0x7FFFFFFF))


def _neg_count_lt(x, cand):
    return jnp.sum(jax.lax.shift_right_arithmetic(x - cand, 31), axis=-1,
                   keepdims=True)


def _topk_thresholds(key, k):
    rows, n = key.shape

    def step(i, t):
        bit = jax.lax.shift_left(jnp.int32(1), jnp.int32(_KEY_BITS - 1) - i)
        cand = t + bit
        cnt = _neg_count_lt(key, cand) + n
        return jnp.where(cnt >= k, cand, t)

    t0 = jnp.full((rows, 1), _KEY_BASE, jnp.int32)
    t = jax.lax.fori_loop(0, _KEY_BITS, step, t0)

    n_gt = _neg_count_lt(key, t + 1) + n
    m = k - n_gt

    idx = jax.lax.broadcasted_iota(jnp.int32, key.shape, 1)
    z = jnp.where(key == t, idx, jnp.int32(2 * n))

    def jstep(i, j):
        bit = jax.lax.shift_left(jnp.int32(1), jnp.int32(13) - i)
        cand = j + bit
        cnt = -_neg_count_lt(z, cand + 1)
        return jnp.where(cnt <= m, cand, j)

    j0 = jnp.full((rows, 1), jnp.int32(-1))
    j = jax.lax.fori_loop(0, 14, jstep, j0)
    return t, z, j


def _body(uw_ref, ue_ref, ks_ref, kt_ref, dr_ref, src_ref, tgt_ref,
          drout_ref):
    rb = ue_ref.shape[0]
    ue = ue_ref[...]
    f_src = jnp.log(uw_ref[0]) + jnp.log(ue) * (1.0 / _EVENT_ALPHA)
    f_tgt = jnp.log(uw_ref[1]) + jnp.log(1.0 - ue) * (1.0 / _EVENT_ALPHA)
    key = jnp.concatenate([_float_key(f_src), _float_key(f_tgt)], axis=0)
    k = jnp.concatenate([ks_ref[:, :1], kt_ref[:, :1]], axis=0)
    t, z, j = _topk_thresholds(key, k)
    mask = (key > t) | (z <= j)
    src_ref[...] = mask[:rb]
    tgt_ref[...] = mask[rb:]
    drout_ref[...] = jnp.broadcast_to(dr_ref[:, :1], drout_ref.shape)


_SC_N = 8192
_SC_BASE = 535567946
_SC_BITS = 30


def _sc_body(uw, ue, kv_hbm, dr_hbm, src_out, tgt_out, dr_out,
             abufs, ebufs, key_buf, out_buf, dr_buf, kv, drv,
             in_sems, out_sems):
    wid = jax.lax.axis_index("s") * 2 + jax.lax.axis_index("c")
    r0 = wid * 2
    rows = (r0, r0 + 1)
    pltpu.sync_copy(kv_hbm, kv)
    pltpu.sync_copy(dr_hbm, drv)

    tasks = []
    for q, (is_src, row) in enumerate(
            [(True, rows[0]), (True, rows[1]),
             (False, rows[0]), (False, rows[1])]):
        tasks.append((is_src, row, abufs[q], ebufs[q],
                      src_out if is_src else tgt_out))

    handles = []
    for q, (is_src, row, a_buf, e_buf, _) in enumerate(tasks):
        handles.append(pltpu.async_copy(
            uw.at[0 if is_src else 1, row], a_buf, in_sems[2 * q]))
        handles.append(pltpu.async_copy(ue.at[row], e_buf,
                                        in_sems[2 * q + 1]))

    out_handles = []
    for q, (is_src, row, a_buf, e_buf, out_ref) in enumerate(tasks):
        handles[2 * q].wait()
        handles[2 * q + 1].wait()

        def build(it, _, a_buf=a_buf, e_buf=e_buf, is_src=is_src):
            for s in range(8):
                off = it * 128 + s * 16
                a = a_buf[pl.ds(off, 16)]
                e = e_buf[pl.ds(off, 16)]
                bb = e if is_src else 1.0 - e
                g = a * a * bb
                key_buf[pl.ds(off, 16)] = jax.lax.bitcast_convert_type(
                    g, jnp.int32)
            return 0

        jax.lax.fori_loop(0, _SC_N // 128, build, 0)

        kpos = (row if is_src else row + 64) * 16
        kval = kv[pl.ds(kpos, 16)][0]

        def bis(i, t):
            bit = jax.lax.shift_left(jnp.int32(1),
                                     jnp.int32(_SC_BITS - 1) - i)
            cand = t + bit

            def cnt8(it, acc):
                for s in range(8):
                    off = it * 128 + s * 16
                    k16 = key_buf[pl.ds(off, 16)]
                    acc = acc + jax.lax.shift_right_arithmetic(k16 - cand,
                                                               31)
                return acc

            acc = jax.lax.fori_loop(0, _SC_N // 128, cnt8,
                                    jnp.zeros((16,), jnp.int32))
            cnt = _SC_N
            for l in range(16):
                cnt = cnt + acc[l]
            return jax.lax.select(cnt >= kval, cand, t)

        t = jax.lax.fori_loop(0, _SC_BITS, bis, jnp.int32(_SC_BASE))

        def mask(it, _):
            for s in range(8):
                off = it * 128 + s * 16
                k16 = key_buf[pl.ds(off, 16)]
                out_buf[pl.ds(off, 16)] = jnp.where(
                    k16 >= t, jnp.int32(1), jnp.int32(0))
            return 0

        jax.lax.fori_loop(0, _SC_N // 128, mask, 0)
        out_handles.append(pltpu.async_copy(out_buf, out_ref.at[row],
                                            out_sems[q]))
        out_handles[-1].wait()

        if is_src:
            dsplat = drv[pl.ds(row * 16, 16)]

            def drfill(it, _, dsplat=dsplat):
                for s in range(8):
                    off = it * 128 + s * 16
                    dr_buf[pl.ds(off, 16)] = dsplat
                return 0

            jax.lax.fori_loop(0, _SC_N // 128, drfill, 0)
            pltpu.sync_copy(dr_buf, dr_out.at[row])


def _sc_call(U_w, U_event, kvec, drcol):
    n = _SC_N
    mesh = plsc.VectorSubcoreMesh(core_axis_name="c", subcore_axis_name="s")
    scratch = (
        [pltpu.VMEM((n,), jnp.float32) for _ in range(4)],
        [pltpu.VMEM((n,), jnp.float32) for _ in range(4)],
        pltpu.VMEM((n,), jnp.int32),
        pltpu.VMEM((n,), jnp.int32),
        pltpu.VMEM((n,), jnp.float32),
        pltpu.VMEM((128 * 16,), jnp.int32),
        pltpu.VMEM((64 * 16,), jnp.float32),
        [pltpu.SemaphoreType.DMA for _ in range(8)],
        [pltpu.SemaphoreType.DMA for _ in range(4)],
    )
    out_type = (
        jax.ShapeDtypeStruct((64, n), jnp.int32),
        jax.ShapeDtypeStruct((64, n), jnp.int32),
        jax.ShapeDtypeStruct((64, n), jnp.float32),
    )
    fn = pl.kernel(_sc_body, out_type=out_type, mesh=mesh,
                   scratch_types=scratch)
    return fn(U_w, U_event, kvec, drcol)


def kernel(U_w, U_event, U_rate):
    b, n = U_event.shape
    lin = jnp.linspace(_EPSILON, 1.0 - _EPSILON, b)
    u = (lin + U_rate) % 1.0
    r_src = jnp.exp(jnp.log(u) / _SRC_ALPHA)
    r_tgt = jnp.exp(jnp.log(1.0 - u) / _TGT_ALPHA)
    dr = jnp.exp(jnp.log(u) * (1.0 / _SRC_ALPHA - 1.0)) / _SRC_ALPHA
    k_src = (r_src * n).astype(jnp.int32)
    k_tgt = (r_tgt * n).astype(jnp.int32)

    kvec = jnp.repeat(jnp.concatenate([k_src, k_tgt]), 16)
    srci, tgti, dr_out = _sc_call(U_w, U_event, kvec, jnp.repeat(dr, 16))
    return (srci.astype(jnp.bool_), tgti.astype(jnp.bool_), dr_out)

    ks = jnp.broadcast_to(k_src[:, None], (b, 128))
    kt = jnp.broadcast_to(k_tgt[:, None], (b, 128))
    drb = jnp.broadcast_to(dr[:, None], (b, 128))

    rb = _ROW_BLOCK
    grid = (b // rb,)
    src, tgt, dr_out = pl.pallas_call(
        _body,
        grid=grid,
        in_specs=[
            pl.BlockSpec((2, rb, n), lambda i: (0, i, 0)),
            pl.BlockSpec((rb, n), lambda i: (i, 0)),
            pl.BlockSpec((rb, 128), lambda i: (i, 0)),
            pl.BlockSpec((rb, 128), lambda i: (i, 0)),
            pl.BlockSpec((rb, 128), lambda i: (i, 0)),
        ],
        out_specs=[
            pl.BlockSpec((rb, n), lambda i: (i, 0)),
            pl.BlockSpec((rb, n), lambda i: (i, 0)),
            pl.BlockSpec((rb, n), lambda i: (i, 0)),
        ],
        out_shape=[
            jax.ShapeDtypeStruct((b, n), jnp.bool_),
            jax.ShapeDtypeStruct((b, n), jnp.bool_),
            jax.ShapeDtypeStruct((b, n), jnp.float32),
        ],
    )(U_w, U_event, ks, kt, drb)
    return (src, tgt, dr_out)


# speedup vs baseline: 1.3327x; 1.3327x over previous
import functools

import jax
import jax.numpy as jnp
from jax.experimental import pallas as pl
from jax.experimental.pallas import tpu as pltpu
from jax.experimental.pallas import tpu_sc as plsc

_EPSILON = 0.05
_SRC_ALPHA = 2.0
_TGT_ALPHA = 2.0
_EVENT_ALPHA = 2.0
_ROW_BLOCK = 64

_KEY_BASE = -1107296257
_KEY_BITS = 28


def _float_key(f):
    b = jax.lax.bitcast_convert_type(f, jnp.int32)
    return b ^ (jnp.right_shift(b, 31) & jnp.int32(0x7FFFFFFF))


def _neg_count_lt(x, cand):
    return jnp.sum(jax.lax.shift_right_arithmetic(x - cand, 31), axis=-1,
                   keepdims=True)


def _topk_thresholds(key, k):
    rows, n = key.shape

    def step(i, t):
        bit = jax.lax.shift_left(jnp.int32(1), jnp.int32(_KEY_BITS - 1) - i)
        cand = t + bit
        cnt = _neg_count_lt(key, cand) + n
        return jnp.where(cnt >= k, cand, t)

    t0 = jnp.full((rows, 1), _KEY_BASE, jnp.int32)
    t = jax.lax.fori_loop(0, _KEY_BITS, step, t0)

    n_gt = _neg_count_lt(key, t + 1) + n
    m = k - n_gt

    idx = jax.lax.broadcasted_iota(jnp.int32, key.shape, 1)
    z = jnp.where(key == t, idx, jnp.int32(2 * n))

    def jstep(i, j):
        bit = jax.lax.shift_left(jnp.int32(1), jnp.int32(13) - i)
        cand = j + bit
        cnt = -_neg_count_lt(z, cand + 1)
        return jnp.where(cnt <= m, cand, j)

    j0 = jnp.full((rows, 1), jnp.int32(-1))
    j = jax.lax.fori_loop(0, 14, jstep, j0)
    return t, z, j


def _body(uw_ref, ue_ref, ks_ref, kt_ref, dr_ref, src_ref, tgt_ref,
          drout_ref):
    rb = ue_ref.shape[0]
    ue = ue_ref[...]
    f_src = jnp.log(uw_ref[0]) + jnp.log(ue) * (1.0 / _EVENT_ALPHA)
    f_tgt = jnp.log(uw_ref[1]) + jnp.log(1.0 - ue) * (1.0 / _EVENT_ALPHA)
    key = jnp.concatenate([_float_key(f_src), _float_key(f_tgt)], axis=0)
    k = jnp.concatenate([ks_ref[:, :1], kt_ref[:, :1]], axis=0)
    t, z, j = _topk_thresholds(key, k)
    mask = (key > t) | (z <= j)
    src_ref[...] = mask[:rb]
    tgt_ref[...] = mask[rb:]
    drout_ref[...] = jnp.broadcast_to(dr_ref[:, :1], drout_ref.shape)


_SC_N = 8192
_SC_BASE = 535567946
_SC_BITS = 30


def _sc_body(uw, ue, kv_hbm, dr_hbm, src_out, tgt_out, dr_out,
             abufs, ebufs, key_buf, out_buf, dr_buf, kv, drv,
             in_sems, out_sems):
    wid = jax.lax.axis_index("s") * 2 + jax.lax.axis_index("c")
    r0 = wid * 2
    rows = (r0, r0 + 1)
    pltpu.sync_copy(kv_hbm, kv)
    pltpu.sync_copy(dr_hbm, drv)

    tasks = []
    for q, (is_src, row) in enumerate(
            [(True, rows[0]), (True, rows[1]),
             (False, rows[0]), (False, rows[1])]):
        tasks.append((is_src, row, abufs[q], ebufs[q],
                      src_out if is_src else tgt_out))

    handles = []
    for q, (is_src, row, a_buf, e_buf, _) in enumerate(tasks):
        handles.append(pltpu.async_copy(
            uw.at[0 if is_src else 1, row], a_buf, in_sems[2 * q]))
        handles.append(pltpu.async_copy(ue.at[row], e_buf,
                                        in_sems[2 * q + 1]))

    out_handles = []
    for q, (is_src, row, a_buf, e_buf, out_ref) in enumerate(tasks):
        handles[2 * q].wait()
        handles[2 * q + 1].wait()

        def build(it, _, a_buf=a_buf, e_buf=e_buf, is_src=is_src):
            for s in range(8):
                off = it * 128 + s * 16
                a = a_buf[pl.ds(off, 16)]
                e = e_buf[pl.ds(off, 16)]
                bb = e if is_src else 1.0 - e
                g = a * a * bb
                key_buf[pl.ds(off, 16)] = jax.lax.bitcast_convert_type(
                    g, jnp.int32)
            return 0

        jax.lax.fori_loop(0, _SC_N // 128, build, 0)

        kpos = (row if is_src else row + 64) * 16
        kval = kv[pl.ds(kpos, 16)][0]

        def bis(i, t):
            bit = jax.lax.shift_left(jnp.int32(1),
                                     jnp.int32(_SC_BITS - 1) - i)
            cand = t + bit

            def cnt8(it, acc):
                for s in range(8):
                    off = it * 128 + s * 16
                    k16 = key_buf[pl.ds(off, 16)]
                    acc = acc + jax.lax.shift_right_arithmetic(k16 - cand,
                                                               31)
                return acc

            acc = jax.lax.fori_loop(0, _SC_N // 128, cnt8,
                                    jnp.zeros((16,), jnp.int32))
            cnt = _SC_N
            for l in range(16):
                cnt = cnt + acc[l]
            return jax.lax.select(cnt >= kval, cand, t)

        t = jax.lax.fori_loop(0, _SC_BITS, bis, jnp.int32(_SC_BASE))

        def mask(it, _):
            for s in range(8):
                off = it * 128 + s * 16
                k16 = key_buf[pl.ds(off, 16)]
                out_buf[pl.ds(off, 16)] = jnp.where(
                    k16 >= t, jnp.int32(1), jnp.int32(0))
            return 0

        jax.lax.fori_loop(0, _SC_N // 128, mask, 0)
        out_handles.append(pltpu.async_copy(out_buf, out_ref.at[row],
                                            out_sems[q]))
        out_handles[-1].wait()

        if is_src:
            dsplat = drv[pl.ds(row * 16, 16)]

            def drfill(it, _, dsplat=dsplat):
                for s in range(8):
                    off = it * 128 + s * 16
                    dr_buf[pl.ds(off, 16)] = dsplat
                return 0

            jax.lax.fori_loop(0, _SC_N // 128, drfill, 0)
            pltpu.sync_copy(dr_buf, dr_out.at[row])


_TC_ROWS = 48


def _sc_body_h(uw, ue, kv_hbm, dr_hbm, mask_out, dr_out,
               a_buf, e_buf, key_buf, out_buf, dr_buf, kv, drv,
               in_sems, out_sem):
    wid = jax.lax.axis_index("s") * 2 + jax.lax.axis_index("c")
    is_src = 1 - (wid & 1)
    row_off = jax.lax.shift_right_logical(wid, 1)
    row = _TC_ROWS + row_off
    pltpu.sync_copy(kv_hbm, kv)
    pltpu.sync_copy(dr_hbm, drv)

    h_a = pltpu.async_copy(uw.at[1 - is_src, row], a_buf, in_sems[0])
    h_e = pltpu.async_copy(ue.at[row], e_buf, in_sems[1])
    h_a.wait()
    h_e.wait()

    is_f = is_src.astype(jnp.float32)
    sgn = 2.0 * is_f - 1.0
    offs = 1.0 - is_f

    def build(it, _):
        for s in range(8):
            off = it * 128 + s * 16
            a = a_buf[pl.ds(off, 16)]
            e = e_buf[pl.ds(off, 16)]
            g = a * a * (offs + sgn * e)
            key_buf[pl.ds(off, 16)] = jax.lax.bitcast_convert_type(
                g, jnp.int32)
        return 0

    jax.lax.fori_loop(0, _SC_N // 128, build, 0)

    kpos = (row + (1 - is_src) * 64) * 16
    kval = kv[pl.ds(kpos, 16)][0]

    def bis(i, t):
        bit = jax.lax.shift_left(jnp.int32(1), jnp.int32(_SC_BITS - 1) - i)
        cand = t + bit

        def cnt8(it, acc):
            for s in range(8):
                off = it * 128 + s * 16
                k16 = key_buf[pl.ds(off, 16)]
                acc = acc + jax.lax.shift_right_arithmetic(k16 - cand, 31)
            return acc

        acc = jax.lax.fori_loop(0, _SC_N // 128, cnt8,
                                jnp.zeros((16,), jnp.int32))
        cnt = _SC_N
        for l in range(16):
            cnt = cnt + acc[l]
        return jax.lax.select(cnt >= kval, cand, t)

    t = jax.lax.fori_loop(0, _SC_BITS, bis, jnp.int32(_SC_BASE))

    def mask(it, _):
        for s in range(8):
            off = it * 128 + s * 16
            k16 = key_buf[pl.ds(off, 16)]
            out_buf[pl.ds(off, 16)] = jnp.where(
                k16 >= t, jnp.int32(1), jnp.int32(0))
        return 0

    jax.lax.fori_loop(0, _SC_N // 128, mask, 0)
    vloc = row_off + (1 - is_src) * 16
    pltpu.async_copy(out_buf, mask_out.at[vloc], out_sem).wait()

    @pl.when(is_src == 1)
    def _():
        dsplat = drv[pl.ds(row * 16, 16)]

        def drfill(it, _):
            for s in range(8):
                off = it * 128 + s * 16
                dr_buf[pl.ds(off, 16)] = dsplat
            return 0

        jax.lax.fori_loop(0, _SC_N // 128, drfill, 0)
        pltpu.sync_copy(dr_buf, dr_out.at[row_off])


def _sc_call_h(U_w, U_event, kvec, drcol):
    n = _SC_N
    sc_rows = 64 - _TC_ROWS
    mesh = plsc.VectorSubcoreMesh(core_axis_name="c", subcore_axis_name="s")
    scratch = (
        pltpu.VMEM((n,), jnp.float32),
        pltpu.VMEM((n,), jnp.float32),
        pltpu.VMEM((n,), jnp.int32),
        pltpu.VMEM((n,), jnp.int32),
        pltpu.VMEM((n,), jnp.float32),
        pltpu.VMEM((128 * 16,), jnp.int32),
        pltpu.VMEM((64 * 16,), jnp.float32),
        [pltpu.SemaphoreType.DMA for _ in range(2)],
        pltpu.SemaphoreType.DMA,
    )
    out_type = (
        jax.ShapeDtypeStruct((2 * sc_rows, n), jnp.int32),
        jax.ShapeDtypeStruct((sc_rows, n), jnp.float32),
    )
    fn = pl.kernel(_sc_body_h, out_type=out_type, mesh=mesh,
                   scratch_types=scratch)
    return fn(U_w, U_event, kvec, drcol)


def _sc_call(U_w, U_event, kvec, drcol):
    n = _SC_N
    mesh = plsc.VectorSubcoreMesh(core_axis_name="c", subcore_axis_name="s")
    scratch = (
        [pltpu.VMEM((n,), jnp.float32) for _ in range(4)],
        [pltpu.VMEM((n,), jnp.float32) for _ in range(4)],
        pltpu.VMEM((n,), jnp.int32),
        pltpu.VMEM((n,), jnp.int32),
        pltpu.VMEM((n,), jnp.float32),
        pltpu.VMEM((128 * 16,), jnp.int32),
        pltpu.VMEM((64 * 16,), jnp.float32),
        [pltpu.SemaphoreType.DMA for _ in range(8)],
        [pltpu.SemaphoreType.DMA for _ in range(4)],
    )
    out_type = (
        jax.ShapeDtypeStruct((64, n), jnp.int32),
        jax.ShapeDtypeStruct((64, n), jnp.int32),
        jax.ShapeDtypeStruct((64, n), jnp.float32),
    )
    fn = pl.kernel(_sc_body, out_type=out_type, mesh=mesh,
                   scratch_types=scratch)
    return fn(U_w, U_event, kvec, drcol)


def kernel(U_w, U_event, U_rate):
    b, n = U_event.shape
    lin = jnp.linspace(_EPSILON, 1.0 - _EPSILON, b)
    u = (lin + U_rate) % 1.0
    r_src = jnp.exp(jnp.log(u) / _SRC_ALPHA)
    r_tgt = jnp.exp(jnp.log(1.0 - u) / _TGT_ALPHA)
    dr = jnp.exp(jnp.log(u) * (1.0 / _SRC_ALPHA - 1.0)) / _SRC_ALPHA
    k_src = (r_src * n).astype(jnp.int32)
    k_tgt = (r_tgt * n).astype(jnp.int32)

    tcr = _TC_ROWS
    ks = jnp.broadcast_to(k_src[:tcr, None], (tcr, 128))
    kt = jnp.broadcast_to(k_tgt[:tcr, None], (tcr, 128))
    drb = jnp.broadcast_to(dr[:tcr, None], (tcr, 128))

    kvec = jnp.repeat(jnp.concatenate([k_src, k_tgt]), 16)
    sc_mask, sc_dr = _sc_call_h(U_w, U_event, kvec, jnp.repeat(dr, 16))

    src, tgt, dr_out = pl.pallas_call(
        _body,
        grid=(1,),
        in_specs=[
            pl.BlockSpec((2, tcr, n), lambda i: (0, 0, 0)),
            pl.BlockSpec((tcr, n), lambda i: (0, 0)),
            pl.BlockSpec((tcr, 128), lambda i: (0, 0)),
            pl.BlockSpec((tcr, 128), lambda i: (0, 0)),
            pl.BlockSpec((tcr, 128), lambda i: (0, 0)),
        ],
        out_specs=[
            pl.BlockSpec((tcr, n), lambda i: (0, 0)),
            pl.BlockSpec((tcr, n), lambda i: (0, 0)),
            pl.BlockSpec((tcr, n), lambda i: (0, 0)),
        ],
        out_shape=[
            jax.ShapeDtypeStruct((tcr, n), jnp.bool_),
            jax.ShapeDtypeStruct((tcr, n), jnp.bool_),
            jax.ShapeDtypeStruct((tcr, n), jnp.float32),
        ],
    )(U_w, U_event, ks, kt, drb)

    scr = 64 - tcr
    src_full = jnp.concatenate([src, sc_mask[:scr].astype(jnp.bool_)], 0)
    tgt_full = jnp.concatenate([tgt, sc_mask[scr:].astype(jnp.bool_)], 0)
    dr_full = jnp.concatenate([dr_out, sc_dr], 0)
    return (src_full, tgt_full, dr_full)


# speedup vs baseline: 1.7800x; 1.3356x over previous
import jax
import jax.numpy as jnp
from jax.experimental import pallas as pl

_EPSILON = 0.05
_SRC_ALPHA = 2.0
_TGT_ALPHA = 2.0
_EVENT_ALPHA = 2.0
_ROW_BLOCK = 64

_KEY_BASE = -1107296257
_KEY_BITS = 28


def _float_key(f):
    b = jax.lax.bitcast_convert_type(f, jnp.int32)
    return b ^ (jnp.right_shift(b, 31) & jnp.int32(0x7FFFFFFF))


def _neg_count_lt(x, cand):
    return jnp.sum(jax.lax.shift_right_arithmetic(x - cand, 31), axis=-1,
                   keepdims=True)


def _topk_thresholds(key, k):
    rows, n = key.shape

    def step(i, t):
        bit = jax.lax.shift_left(jnp.int32(1), jnp.int32(_KEY_BITS - 1) - i)
        cand = t + bit
        cnt = _neg_count_lt(key, cand) + n
        return jnp.where(cnt >= k, cand, t)

    t0 = jnp.full((rows, 1), _KEY_BASE, jnp.int32)
    t = jax.lax.fori_loop(0, _KEY_BITS, step, t0)

    n_gt = _neg_count_lt(key, t + 1) + n
    m = k - n_gt

    idx = jax.lax.broadcasted_iota(jnp.int32, key.shape, 1)
    z = jnp.where(key == t, idx, jnp.int32(2 * n))

    def jstep(i, j):
        bit = jax.lax.shift_left(jnp.int32(1), jnp.int32(13) - i)
        cand = j + bit
        cnt = -_neg_count_lt(z, cand + 1)
        return jnp.where(cnt <= m, cand, j)

    j0 = jnp.full((rows, 1), jnp.int32(-1))
    j = jax.lax.fori_loop(0, 14, jstep, j0)
    return t, z <= j


def _body(uw_ref, ue_ref, ks_ref, kt_ref, dr_ref, src_ref, tgt_ref,
          drout_ref):
    rb = ue_ref.shape[0]
    ue = ue_ref[...]
    f_src = jnp.log(uw_ref[0]) + jnp.log(ue) * (1.0 / _EVENT_ALPHA)
    f_tgt = jnp.log(uw_ref[1]) + jnp.log(1.0 - ue) * (1.0 / _EVENT_ALPHA)
    key = jnp.concatenate([_float_key(f_src), _float_key(f_tgt)], axis=0)
    k = jnp.concatenate([ks_ref[:, :1], kt_ref[:, :1]], axis=0)
    t, tie_mask = _topk_thresholds(key, k)
    mask = (key > t) | tie_mask
    src_ref[...] = mask[:rb]
    tgt_ref[...] = mask[rb:]
    drout_ref[...] = jnp.broadcast_to(dr_ref[:, :1], drout_ref.shape)


def kernel(U_w, U_event, U_rate):
    b, n = U_event.shape
    lin = jnp.linspace(_EPSILON, 1.0 - _EPSILON, b)
    u = (lin + U_rate) % 1.0
    r_src = jnp.exp(jnp.log(u) / _SRC_ALPHA)
    r_tgt = jnp.exp(jnp.log(1.0 - u) / _TGT_ALPHA)
    dr = jnp.exp(jnp.log(u) * (1.0 / _SRC_ALPHA - 1.0)) / _SRC_ALPHA
    k_src = (r_src * n).astype(jnp.int32)
    k_tgt = (r_tgt * n).astype(jnp.int32)

    ks = jnp.broadcast_to(k_src[:, None], (b, 128))
    kt = jnp.broadcast_to(k_tgt[:, None], (b, 128))
    drb = jnp.broadcast_to(dr[:, None], (b, 128))

    rb = _ROW_BLOCK
    grid = (b // rb,)
    src, tgt, dr_out = pl.pallas_call(
        _body,
        grid=grid,
        in_specs=[
            pl.BlockSpec((2, rb, n), lambda i: (0, i, 0)),
            pl.BlockSpec((rb, n), lambda i: (i, 0)),
            pl.BlockSpec((rb, 128), lambda i: (i, 0)),
            pl.BlockSpec((rb, 128), lambda i: (i, 0)),
            pl.BlockSpec((rb, 128), lambda i: (i, 0)),
        ],
        out_specs=[
            pl.BlockSpec((rb, n), lambda i: (i, 0)),
            pl.BlockSpec((rb, n), lambda i: (i, 0)),
            pl.BlockSpec((rb, n), lambda i: (i, 0)),
        ],
        out_shape=[
            jax.ShapeDtypeStruct((b, n), jnp.bool_),
            jax.ShapeDtypeStruct((b, n), jnp.bool_),
            jax.ShapeDtypeStruct((b, n), jnp.float32),
        ],
    )(U_w, U_event, ks, kt, drb)
    return (src, tgt, dr_out)
